# baseline (speedup 1.0000x reference)
import functools

import jax
import jax.numpy as jnp
from jax import lax
from jax.experimental import pallas as pl
from jax.experimental.pallas import tpu as pltpu
from jax.experimental.pallas import tpu_sc as plsc

NC = 2
NS = 16
NW = NC * NS

BATCH = 16384
NUM_ITEMS = 100000
D_EMB = 123
D_OUT = 128
B_PER_W = BATCH // NW
CH = 32
N_CHUNK = B_PER_W // CH


def _genre_vec(b1_v, w2_v, b2_v, w3_v, b3_v):
    b1a = b1_v[pl.ds(0, 16)]
    b1b = b1_v[pl.ds(16, 16)]
    t0 = b2_v[pl.ds(0, 16)]
    t1 = b2_v[pl.ds(16, 16)]
    for k in range(30):
        bk = b1a[k] if k < 16 else b1b[k - 16]
        t0 = t0 + bk * w2_v[k, pl.ds(0, 16)]
        t1 = t1 + bk * w2_v[k, pl.ds(16, 16)]
    h = b3_v[pl.ds(0, 16)]
    for k in range(30):
        tk = t0[k] if k < 16 else t1[k - 16]
        h = h + tk * w3_v[k, pl.ds(0, 16)]
    return h


def _sc_body(idx_hbm, table_hbm, b1_hbm, w2_hbm, b2_hbm, w3_hbm, b3_hbm,
             out_hbm,
             idx_v, ba_v, bb_v, oa_v, ob_v,
             b1_v, w2_v, b2_v, w3_v, b3_v,
             gsem_a, gsem_b, osem_a, osem_b):
    wid = lax.axis_index("s") * NC + lax.axis_index("c")
    base = wid * B_PER_W
    blocks = (ba_v, bb_v)
    outs = (oa_v, ob_v)
    gsems = (gsem_a, gsem_b)
    osems = (osem_a, osem_b)

    pltpu.sync_copy(idx_hbm.at[wid], idx_v)

    pltpu.sync_copy(b1_hbm, b1_v)
    pltpu.sync_copy(w2_hbm, w2_v)
    pltpu.sync_copy(b2_hbm, b2_v)
    pltpu.sync_copy(w3_hbm, w3_v)
    pltpu.sync_copy(b3_hbm, b3_v)
    h = _genre_vec(b1_v, w2_v, b2_v, w3_v, b3_v)

    for o in outs:
        for r in range(CH):
            o[r, pl.ds(D_OUT - 16, 16)] = h

    def fire_gather(ci, buf):
        copies = []
        for g in range(CH // 16):
            blkv = lax.shift_right_logical(
                idx_v[pl.ds(ci * CH + g * 16, 16)], 3)
            for lane in range(16):
                copies.append(
                    pltpu.async_copy(table_hbm.at[blkv[lane]],
                                     blocks[buf].at[g * 16 + lane],
                                     gsems[buf]))
        return copies

    def extract(ci, buf):
        blk = blocks[buf]
        out_c = outs[buf]
        for g in range(CH // 16):
            subv = lax.bitwise_and(idx_v[pl.ds(ci * CH + g * 16, 16)], 7)
            for lane in range(16):
                i = g * 16 + lane
                sub = subv[lane]
                for c in range(7):
                    csl = pl.ds(c * 16, 16)
                    out_c[i, csl] = blk[i, sub, csl]
                tsl = pl.ds(D_EMB - 16, 16)
                out_c[i, tsl] = blk[i, sub, tsl]
        return pltpu.async_copy(out_c, out_hbm.at[pl.ds(base + ci * CH, CH)],
                                osems[buf])

    def super_step(k, _):
        ca = 2 * k
        cb = 2 * k + 1
        ga = fire_gather(ca, 0)
        gb = fire_gather(cb, 1)
        for c in ga:
            c.wait()
        wa = extract(ca, 0)
        for c in gb:
            c.wait()
        wb = extract(cb, 1)
        wa.wait()
        wb.wait()
        return _

    lax.fori_loop(0, N_CHUNK // 2, super_step, 0)


@jax.jit
def _sc_call(idx2d, table3d, b1p, W2p, b2p, W3p, b3p):
    mesh = plsc.VectorSubcoreMesh(core_axis_name="c", subcore_axis_name="s")
    run = functools.partial(
        pl.kernel,
        out_type=jax.ShapeDtypeStruct((BATCH, D_OUT), jnp.float32),
        mesh=mesh,
        scratch_types=[
            pltpu.VMEM((B_PER_W,), jnp.int32),
            pltpu.VMEM((CH, 8, D_EMB), jnp.float32),
            pltpu.VMEM((CH, 8, D_EMB), jnp.float32),
            pltpu.VMEM((CH, D_OUT), jnp.float32),
            pltpu.VMEM((CH, D_OUT), jnp.float32),
            pltpu.VMEM((32,), jnp.float32),
            pltpu.VMEM((30, 32), jnp.float32),
            pltpu.VMEM((32,), jnp.float32),
            pltpu.VMEM((30, 16), jnp.float32),
            pltpu.VMEM((16,), jnp.float32),
            pltpu.SemaphoreType.DMA,
            pltpu.SemaphoreType.DMA,
            pltpu.SemaphoreType.DMA,
            pltpu.SemaphoreType.DMA,
        ],
    )(_sc_body)
    return run(idx2d, table3d, b1p, W2p, b2p, W3p, b3p)


def kernel(item_inputs, W_emb, W1, b1, W2, b2, W3, b3):
    del W1
    idx2d = item_inputs.reshape(NW, B_PER_W)
    table3d = W_emb.reshape(NUM_ITEMS // 8, 8, D_EMB)
    b1p = jnp.pad(b1, (0, 2))
    W2p = jnp.pad(W2, ((0, 0), (0, 2)))
    b2p = jnp.pad(b2, (0, 2))
    W3p = jnp.pad(W3, ((0, 0), (11, 0)))
    b3p = jnp.pad(b3, (11, 0))
    return _sc_call(idx2d, table3d, b1p, W2p, b2p, W3p, b3p)


# speedup vs baseline: 1.2670x; 1.2670x over previous
import functools

import jax
import jax.numpy as jnp
from jax import lax
from jax.experimental import pallas as pl
from jax.experimental.pallas import tpu as pltpu
from jax.experimental.pallas import tpu_sc as plsc

NC = 2
NS = 16
NW = NC * NS

BATCH = 16384
D_EMB = 123
D_OUT = 128
B_PER_W = BATCH // NW
N_CHUNK = B_PER_W // 128


def _genre_vec(pw_v):
    b1a = pw_v[0, pl.ds(0, 16)]
    b1b = pw_v[0, pl.ds(16, 16)]
    t0 = pw_v[31, pl.ds(0, 16)]
    t1 = pw_v[31, pl.ds(16, 16)]
    for k in range(30):
        bk = b1a[k] if k < 16 else b1b[k - 16]
        t0 = t0 + bk * pw_v[1 + k, pl.ds(0, 16)]
        t1 = t1 + bk * pw_v[1 + k, pl.ds(16, 16)]
    h = pw_v[32, pl.ds(32, 16)]
    for k in range(30):
        tk = t0[k] if k < 16 else t1[k - 16]
        h = h + tk * pw_v[1 + k, pl.ds(32, 16)]
    return h


def _sc_body(idx_hbm, table_hbm, pw_hbm,
             out_hbm,
             idx_v, out_v, pw_v,
             gsem_0, gsem_1, gsem_2, gsem_3, wsem, osem):
    wid = lax.axis_index("s") * NC + lax.axis_index("c")
    base = wid * B_PER_W
    gsems = (gsem_0, gsem_1, gsem_2, gsem_3)

    pltpu.sync_copy(idx_hbm.at[wid], idx_v)
    wcopy = pltpu.async_copy(pw_hbm, pw_v, wsem)
    gathers = []
    for j in range(N_CHUNK):
        sl = pl.ds(j * 128, 128)
        gathers.append(
            pltpu.async_copy(table_hbm.at[idx_v.at[sl]], out_v.at[sl],
                             gsems[j]))

    wcopy.wait()
    h = _genre_vec(pw_v)

    writes = []
    for j in range(N_CHUNK):
        gathers[j].wait()
        for r in range(j * 128, (j + 1) * 128):
            sl = (r, pl.ds(D_OUT - 16, 16))
            out_v[sl] = out_v[sl] + h
        csl = pl.ds(j * 128, 128)
        writes.append(
            pltpu.async_copy(out_v.at[csl],
                             out_hbm.at[pl.ds(base + j * 128, 128)], osem))
    for w in writes:
        w.wait()


@jax.jit
def _sc_call(idx2d, table_pad, pw):
    mesh = plsc.VectorSubcoreMesh(core_axis_name="c", subcore_axis_name="s")
    run = functools.partial(
        pl.kernel,
        out_type=jax.ShapeDtypeStruct((BATCH, D_OUT), jnp.float32),
        mesh=mesh,
        scratch_types=[
            pltpu.VMEM((B_PER_W,), jnp.int32),
            pltpu.VMEM((B_PER_W, D_OUT), jnp.float32),
            pltpu.VMEM((33, 48), jnp.float32),
            pltpu.SemaphoreType.DMA,
            pltpu.SemaphoreType.DMA,
            pltpu.SemaphoreType.DMA,
            pltpu.SemaphoreType.DMA,
            pltpu.SemaphoreType.DMA,
            pltpu.SemaphoreType.DMA,
        ],
    )(_sc_body)
    return run(idx2d, table_pad, pw)


def kernel(item_inputs, W_emb, W1, b1, W2, b2, W3, b3):
    del W1
    idx2d = item_inputs.reshape(NW, B_PER_W)
    table_pad = jnp.pad(W_emb, ((0, 0), (0, D_OUT - D_EMB)))
    pw = jnp.zeros((33, 48), jnp.float32)
    pw = pw.at[0, :30].set(b1)
    pw = pw.at[1:31, :30].set(W2)
    pw = pw.at[31, :30].set(b2)
    pw = pw.at[1:31, 43:48].set(W3)
    pw = pw.at[32, 43:48].set(b3)
    return _sc_call(idx2d, table_pad, pw)


# speedup vs baseline: 1.2904x; 1.0185x over previous
import functools

import jax
import jax.numpy as jnp
from jax import lax
from jax.experimental import pallas as pl
from jax.experimental.pallas import tpu as pltpu
from jax.experimental.pallas import tpu_sc as plsc

NC = 2
NS = 16
NW = NC * NS

BATCH = 16384
D_EMB = 123
D_OUT = 128
B_PER_W = BATCH // NW
N_CHUNK = B_PER_W // 128


def _genre_vec(pw_v):
    b1a = pw_v[0, pl.ds(0, 16)]
    b1b = pw_v[0, pl.ds(16, 16)]
    t0 = pw_v[31, pl.ds(0, 16)]
    t1 = pw_v[31, pl.ds(16, 16)]
    for k in range(30):
        bk = b1a[k] if k < 16 else b1b[k - 16]
        t0 = t0 + bk * pw_v[1 + k, pl.ds(0, 16)]
        t1 = t1 + bk * pw_v[1 + k, pl.ds(16, 16)]
    h = pw_v[32, pl.ds(32, 16)]
    for k in range(30):
        tk = t0[k] if k < 16 else t1[k - 16]
        h = h + tk * pw_v[1 + k, pl.ds(32, 16)]
    return h


def _sc_body(idx_hbm, table_hbm, pw_hbm,
             out_hbm,
             idx_v, out_v, pw_v,
             gsem_0, gsem_1, gsem_2, gsem_3, wsem, osem):
    wid = lax.axis_index("s") * NC + lax.axis_index("c")
    base = wid * B_PER_W
    gsems = (gsem_0, gsem_1, gsem_2, gsem_3)

    pltpu.sync_copy(idx_hbm.at[pl.ds(base, B_PER_W)], idx_v)
    wcopy = pltpu.async_copy(pw_hbm, pw_v, wsem)
    gathers = []
    for j in range(N_CHUNK):
        sl = pl.ds(j * 128, 128)
        gathers.append(
            pltpu.async_copy(table_hbm.at[idx_v.at[sl]], out_v.at[sl],
                             gsems[j]))

    wcopy.wait()
    h = _genre_vec(pw_v)

    writes = []
    for j in range(N_CHUNK):
        gathers[j].wait()
        for r in range(j * 128, (j + 1) * 128):
            plsc.addupdate(out_v.at[r, pl.ds(D_OUT - 16, 16)], h)
        csl = pl.ds(j * 128, 128)
        writes.append(
            pltpu.async_copy(out_v.at[csl],
                             out_hbm.at[pl.ds(base + j * 128, 128)], osem))
    for w in writes:
        w.wait()


@jax.jit
def _sc_call(idx, table_pad, pw):
    mesh = plsc.VectorSubcoreMesh(core_axis_name="c", subcore_axis_name="s")
    run = functools.partial(
        pl.kernel,
        out_type=jax.ShapeDtypeStruct((BATCH, D_OUT), jnp.float32),
        mesh=mesh,
        scratch_types=[
            pltpu.VMEM((B_PER_W,), jnp.int32),
            pltpu.VMEM((B_PER_W, D_OUT), jnp.float32),
            pltpu.VMEM((33, 48), jnp.float32),
            pltpu.SemaphoreType.DMA,
            pltpu.SemaphoreType.DMA,
            pltpu.SemaphoreType.DMA,
            pltpu.SemaphoreType.DMA,
            pltpu.SemaphoreType.DMA,
            pltpu.SemaphoreType.DMA,
        ],
    )(_sc_body)
    return run(idx, table_pad, pw)


def kernel(item_inputs, W_emb, W1, b1, W2, b2, W3, b3):
    del W1
    table_pad = jnp.pad(W_emb, ((0, 0), (0, D_OUT - D_EMB)))
    mid = (jnp.pad(W2, ((0, 0), (0, 18)))
           + jnp.pad(W3, ((0, 0), (43, 0))))
    pw = jnp.concatenate([
        jnp.pad(b1, (0, 18))[None],
        mid,
        jnp.pad(b2, (0, 18))[None],
        jnp.pad(b3, (43, 0))[None],
    ], axis=0)
    return _sc_call(item_inputs, table_pad, pw)


# speedup vs baseline: 1.3394x; 1.0380x over previous
import functools

import jax
import jax.numpy as jnp
from jax import lax
from jax.experimental import pallas as pl
from jax.experimental.pallas import tpu as pltpu
from jax.experimental.pallas import tpu_sc as plsc

NC = 2
NS = 16
NW = NC * NS

BATCH = 16384
D_EMB = 123
D_OUT = 128
B_PER_W = BATCH // NW
N_CHUNK = B_PER_W // 128


def _h_body(b1_ref, w2_ref, b2_ref, w3_ref, b3_ref, h_ref):
    t = b1_ref[:].reshape(1, 30) @ w2_ref[:] + b2_ref[:].reshape(1, 30)
    h_ref[:] = (t @ w3_ref[:] + b3_ref[:].reshape(1, 5)).reshape(5)


@jax.jit
def _h_call(b1, W2, b2, W3, b3):
    return pl.pallas_call(
        _h_body,
        out_shape=jax.ShapeDtypeStruct((5,), jnp.float32),
    )(b1, W2, b2, W3, b3)


def _sc_body(idx_hbm, table_hbm, out_hbm, idx_v, out_v,
             gsem_0, gsem_1, gsem_2, gsem_3, osem):
    wid = lax.axis_index("s") * NC + lax.axis_index("c")
    base = wid * B_PER_W
    gsems = (gsem_0, gsem_1, gsem_2, gsem_3)

    pltpu.sync_copy(idx_hbm.at[pl.ds(base, B_PER_W)], idx_v)
    gathers = []
    for j in range(N_CHUNK):
        sl = pl.ds(j * 128, 128)
        gathers.append(
            pltpu.async_copy(table_hbm.at[idx_v.at[sl]], out_v.at[sl],
                             gsems[j]))

    writes = []
    for j in range(N_CHUNK):
        gathers[j].wait()
        csl = pl.ds(j * 128, 128)
        writes.append(
            pltpu.async_copy(out_v.at[csl],
                             out_hbm.at[pl.ds(base + j * 128, 128)], osem))
    for w in writes:
        w.wait()


@jax.jit
def _sc_call(idx, table_pad):
    mesh = plsc.VectorSubcoreMesh(core_axis_name="c", subcore_axis_name="s")
    run = functools.partial(
        pl.kernel,
        out_type=jax.ShapeDtypeStruct((BATCH, D_OUT), jnp.float32),
        mesh=mesh,
        scratch_types=[
            pltpu.VMEM((B_PER_W,), jnp.int32),
            pltpu.VMEM((B_PER_W, D_OUT), jnp.float32),
            pltpu.SemaphoreType.DMA,
            pltpu.SemaphoreType.DMA,
            pltpu.SemaphoreType.DMA,
            pltpu.SemaphoreType.DMA,
            pltpu.SemaphoreType.DMA,
        ],
    )(_sc_body)
    return run(idx, table_pad)


def kernel(item_inputs, W_emb, W1, b1, W2, b2, W3, b3):
    del W1
    h = _h_call(b1, W2, b2, W3, b3)
    table_pad = jnp.concatenate(
        [W_emb, jnp.broadcast_to(h, (W_emb.shape[0], 5))], axis=1)
    return _sc_call(item_inputs, table_pad)
